# baseline (speedup 1.0000x reference)
import functools

import jax
import jax.numpy as jnp
from jax import lax
from jax.experimental import pallas as pl
from jax.experimental.pallas import tpu as pltpu
from jax.experimental.pallas import tpu_sc as plsc

N = 10000
D = 128
NC = 2
NS = 16
NW = NC * NS
CHUNK = 128
CPT = 80
EPT = CPT * CHUNK
E2 = NW * EPT
PCH = 16
CPT_A = 112
CPT_B = 48
NPAD = 10240
STRIPE = NPAD // NS
PAD_BIN = 10200

_mesh = plsc.VectorSubcoreMesh(
    core_axis_name="c", subcore_axis_name="s", num_cores=NC, num_subcores=NS)


@functools.partial(
    pl.kernel,
    out_type=[jax.ShapeDtypeStruct((NPAD,), jnp.float32) for _ in range(NC)],
    mesh=_mesh,
    scratch_types=[
        pltpu.VMEM((CPT, CHUNK), jnp.int32),
        pltpu.VMEM((CHUNK,), jnp.float32),
        pltpu.VMEM((STRIPE,), jnp.float32),
        pltpu.VMEM_SHARED((NPAD,), jnp.float32),
    ],
)
def _sc_deg(dst_hbm, d0_hbm, d1_hbm, didx, ones_v, zrow, deg_sp):
    c = lax.axis_index("c")
    s = lax.axis_index("s")
    w = c * NS + s
    for i in range(CHUNK // 16):
        ones_v[pl.ds(i * 16, 16)] = jnp.ones((16,), jnp.float32)
    for i in range(STRIPE // 16):
        zrow[pl.ds(i * 16, 16)] = jnp.zeros((16,), jnp.float32)
    pltpu.sync_copy(zrow, deg_sp.at[pl.ds(s * STRIPE, STRIPE)])
    pltpu.sync_copy(dst_hbm.at[pl.ds(w * CPT, CPT)], didx)
    plsc.subcore_barrier()

    def body(j, carry):
        pltpu.sync_copy(ones_v, deg_sp.at[didx.at[j]], add=True)
        return carry

    lax.fori_loop(0, CPT, body, 0)
    plsc.subcore_barrier()

    @pl.when(c == 0)
    def _():
        pltpu.sync_copy(deg_sp.at[pl.ds(s * STRIPE, STRIPE)],
                        d0_hbm.at[pl.ds(s * STRIPE, STRIPE)])

    @pl.when(c == 1)
    def _():
        pltpu.sync_copy(deg_sp.at[pl.ds(s * STRIPE, STRIPE)],
                        d1_hbm.at[pl.ds(s * STRIPE, STRIPE)])


@functools.partial(
    pl.kernel,
    out_type=[jax.ShapeDtypeStruct((NPAD, D), jnp.float32) for _ in range(NC)],
    mesh=_mesh,
    scratch_types=[
        pltpu.VMEM((PCH, CHUNK), jnp.int32),
        pltpu.VMEM((PCH, CHUNK), jnp.int32),
        pltpu.VMEM((CHUNK, D), jnp.float32),
        pltpu.VMEM((CHUNK, D), jnp.float32),
        pltpu.VMEM_SHARED((NPAD, D), jnp.float32),
        pltpu.SemaphoreType.DMA,
        pltpu.SemaphoreType.DMA,
        pltpu.SemaphoreType.DMA,
        pltpu.SemaphoreType.DMA,
    ],
)
def _sc_edge(src_hbm, dst_hbm, g_hbm, z_hbm, a0_hbm, a1_hbm,
             sidx, didx, rows0, rows1, acc_sp, sem0, sem1, sem2, sem3):
    c = lax.axis_index("c")
    s = lax.axis_index("s")
    w = c * NS + s
    for i in range(STRIPE // CHUNK):
        pltpu.sync_copy(z_hbm, acc_sp.at[pl.ds(s * STRIPE + i * CHUNK, CHUNK)])
    plsc.subcore_barrier()

    nph = jnp.where(c == 0, CPT_A // PCH, CPT_B // PCH)
    cbase = jnp.where(c == 0, s * CPT_A, NS * CPT_A + s * CPT_B)

    def phase(p, pcarry):
        base = pl.multiple_of(cbase + p * PCH, 8)
        pltpu.sync_copy(src_hbm.at[pl.ds(base, PCH)], sidx)
        pltpu.sync_copy(dst_hbm.at[pl.ds(base, PCH)], didx)
        pltpu.async_copy(g_hbm.at[sidx.at[0]], rows0, sem0)
        pltpu.async_copy(g_hbm.at[sidx.at[1]], rows1, sem1)

        def body(i, carry):
            j0 = 2 * i
            pltpu.make_async_copy(g_hbm.at[sidx.at[j0]], rows0, sem0).wait()
            pltpu.async_copy(rows0, acc_sp.at[didx.at[j0]], sem2, add=True)
            pltpu.make_async_copy(g_hbm.at[sidx.at[j0 + 1]], rows1, sem1).wait()
            pltpu.async_copy(rows1, acc_sp.at[didx.at[j0 + 1]], sem3, add=True)

            @pl.when(j0 + 2 < PCH)
            def _():
                pltpu.make_async_copy(rows0, acc_sp.at[didx.at[j0]],
                                      sem2).wait()
                pltpu.async_copy(g_hbm.at[sidx.at[j0 + 2]], rows0, sem0)
                pltpu.make_async_copy(rows1, acc_sp.at[didx.at[j0 + 1]],
                                      sem3).wait()
                pltpu.async_copy(g_hbm.at[sidx.at[j0 + 3]], rows1, sem1)

            return carry

        lax.fori_loop(0, PCH // 2, body, 0)
        pltpu.make_async_copy(rows0, acc_sp.at[didx.at[PCH - 2]], sem2).wait()
        pltpu.make_async_copy(rows1, acc_sp.at[didx.at[PCH - 1]], sem3).wait()
        return pcarry

    lax.fori_loop(0, nph, phase, 0)
    plsc.subcore_barrier()

    @pl.when(c == 0)
    def _():
        for i in range(STRIPE // CHUNK):
            r0 = s * STRIPE + i * CHUNK
            pltpu.sync_copy(acc_sp.at[pl.ds(r0, CHUNK)],
                            a0_hbm.at[pl.ds(r0, CHUNK)])

    @pl.when(c == 1)
    def _():
        for i in range(STRIPE // CHUNK):
            r0 = s * STRIPE + i * CHUNK
            pltpu.sync_copy(acc_sp.at[pl.ds(r0, CHUNK)],
                            a1_hbm.at[pl.ds(r0, CHUNK)])


_BLK = 1000


def _tc_linear_body(x_ref, w_ref, b_ref, deg_ref, g_ref):
    h = jnp.dot(x_ref[...], w_ref[...], preferred_element_type=jnp.float32)
    h = h + b_ref[...]
    g_ref[...] = h * lax.rsqrt(deg_ref[...])


def _tc_finish_body(a0_ref, a1_ref, g_ref, deg_ref, out_ref):
    tot = a0_ref[...] + a1_ref[...] + g_ref[...]
    out_ref[...] = jnp.maximum(lax.rsqrt(deg_ref[...]) * tot, 0.0)


_tc_linear = pl.pallas_call(
    _tc_linear_body,
    grid=(N // _BLK,),
    in_specs=[
        pl.BlockSpec((_BLK, D), lambda i: (i, 0)),
        pl.BlockSpec((D, D), lambda i: (0, 0)),
        pl.BlockSpec((1, D), lambda i: (0, 0)),
        pl.BlockSpec((_BLK, 1), lambda i: (i, 0)),
    ],
    out_specs=pl.BlockSpec((_BLK, D), lambda i: (i, 0)),
    out_shape=jax.ShapeDtypeStruct((N, D), jnp.float32),
)

_tc_finish = pl.pallas_call(
    _tc_finish_body,
    grid=(N // _BLK,),
    in_specs=[
        pl.BlockSpec((_BLK, D), lambda i: (i, 0)),
        pl.BlockSpec((_BLK, D), lambda i: (i, 0)),
        pl.BlockSpec((_BLK, D), lambda i: (i, 0)),
        pl.BlockSpec((_BLK, 1), lambda i: (i, 0)),
    ],
    out_specs=pl.BlockSpec((_BLK, D), lambda i: (i, 0)),
    out_shape=jax.ShapeDtypeStruct((N, D), jnp.float32),
)


def kernel(x, edge_index, W, b):
    src = edge_index[0]
    dst = edge_index[1]
    pad = E2 - src.shape[0]
    src_p = jnp.concatenate(
        [src, jnp.zeros((pad,), jnp.int32)]).reshape(E2 // CHUNK, CHUNK)
    dst_p = jnp.concatenate(
        [dst, jnp.full((pad,), PAD_BIN, jnp.int32)]).reshape(E2 // CHUNK, CHUNK)

    d0, d1 = _sc_deg(dst_p)
    deg2 = (d0 + d1)[:N, None] + 1.0

    g = _tc_linear(x, W, b.reshape(1, D), deg2)

    z128 = jnp.zeros((CHUNK, D), jnp.float32)
    a0, a1 = _sc_edge(src_p, dst_p, g, z128)

    return _tc_finish(a0, a1, g, deg2)


# speedup vs baseline: 1.0320x; 1.0320x over previous
import functools

import jax
import jax.numpy as jnp
from jax import lax
from jax.experimental import pallas as pl
from jax.experimental.pallas import tpu as pltpu
from jax.experimental.pallas import tpu_sc as plsc

N = 10000
D = 128
NC = 2
NS = 16
NW = NC * NS
CHUNK = 128
CPT = 80
EPT = CPT * CHUNK
E2 = NW * EPT
PCH = 32
CPT_A = 128
CPT_B = 32
NPAD = 10240
STRIPE = NPAD // NS
PAD_BIN = 10200

_mesh = plsc.VectorSubcoreMesh(
    core_axis_name="c", subcore_axis_name="s", num_cores=NC, num_subcores=NS)


@functools.partial(
    pl.kernel,
    out_type=[jax.ShapeDtypeStruct((NPAD,), jnp.float32) for _ in range(NC)],
    mesh=_mesh,
    scratch_types=[
        pltpu.VMEM((CPT, CHUNK), jnp.int32),
        pltpu.VMEM((CHUNK,), jnp.float32),
        pltpu.VMEM((STRIPE,), jnp.float32),
        pltpu.VMEM_SHARED((NPAD,), jnp.float32),
    ],
)
def _sc_deg(dst_hbm, d0_hbm, d1_hbm, didx, ones_v, zrow, deg_sp):
    c = lax.axis_index("c")
    s = lax.axis_index("s")
    w = c * NS + s
    for i in range(CHUNK // 16):
        ones_v[pl.ds(i * 16, 16)] = jnp.ones((16,), jnp.float32)
    for i in range(STRIPE // 16):
        zrow[pl.ds(i * 16, 16)] = jnp.zeros((16,), jnp.float32)
    pltpu.sync_copy(zrow, deg_sp.at[pl.ds(s * STRIPE, STRIPE)])
    pltpu.sync_copy(dst_hbm.at[pl.ds(w * CPT, CPT)], didx)
    plsc.subcore_barrier()

    def body(j, carry):
        pltpu.sync_copy(ones_v, deg_sp.at[didx.at[j]], add=True)
        return carry

    lax.fori_loop(0, CPT, body, 0)
    plsc.subcore_barrier()

    @pl.when(c == 0)
    def _():
        pltpu.sync_copy(deg_sp.at[pl.ds(s * STRIPE, STRIPE)],
                        d0_hbm.at[pl.ds(s * STRIPE, STRIPE)])

    @pl.when(c == 1)
    def _():
        pltpu.sync_copy(deg_sp.at[pl.ds(s * STRIPE, STRIPE)],
                        d1_hbm.at[pl.ds(s * STRIPE, STRIPE)])


@functools.partial(
    pl.kernel,
    out_type=[jax.ShapeDtypeStruct((NPAD, D), jnp.float32) for _ in range(NC)],
    mesh=_mesh,
    scratch_types=[
        pltpu.VMEM((PCH, CHUNK), jnp.int32),
        pltpu.VMEM((PCH, CHUNK), jnp.int32),
        pltpu.VMEM((CHUNK, D), jnp.float32),
        pltpu.VMEM((CHUNK, D), jnp.float32),
        pltpu.VMEM_SHARED((NPAD, D), jnp.float32),
        pltpu.SemaphoreType.DMA,
        pltpu.SemaphoreType.DMA,
        pltpu.SemaphoreType.DMA,
        pltpu.SemaphoreType.DMA,
    ],
)
def _sc_edge(src_hbm, dst_hbm, g_hbm, z_hbm, a0_hbm, a1_hbm,
             sidx, didx, rows0, rows1, acc_sp, sem0, sem1, sem2, sem3):
    c = lax.axis_index("c")
    s = lax.axis_index("s")
    w = c * NS + s
    for i in range(STRIPE // CHUNK):
        pltpu.sync_copy(z_hbm, acc_sp.at[pl.ds(s * STRIPE + i * CHUNK, CHUNK)])
    plsc.subcore_barrier()

    nph = jnp.where(c == 0, CPT_A // PCH, CPT_B // PCH)
    cbase = jnp.where(c == 0, s * CPT_A, NS * CPT_A + s * CPT_B)

    def phase(p, pcarry):
        base = pl.multiple_of(cbase + p * PCH, 8)
        pltpu.sync_copy(src_hbm.at[pl.ds(base, PCH)], sidx)
        pltpu.sync_copy(dst_hbm.at[pl.ds(base, PCH)], didx)
        pltpu.async_copy(g_hbm.at[sidx.at[0]], rows0, sem0)
        pltpu.async_copy(g_hbm.at[sidx.at[1]], rows1, sem1)

        def body(i, carry):
            j0 = 2 * i
            pltpu.make_async_copy(g_hbm.at[sidx.at[j0]], rows0, sem0).wait()
            pltpu.async_copy(rows0, acc_sp.at[didx.at[j0]], sem2, add=True)
            pltpu.make_async_copy(g_hbm.at[sidx.at[j0 + 1]], rows1, sem1).wait()
            pltpu.async_copy(rows1, acc_sp.at[didx.at[j0 + 1]], sem3, add=True)

            @pl.when(j0 + 2 < PCH)
            def _():
                pltpu.make_async_copy(rows0, acc_sp.at[didx.at[j0]],
                                      sem2).wait()
                pltpu.async_copy(g_hbm.at[sidx.at[j0 + 2]], rows0, sem0)
                pltpu.make_async_copy(rows1, acc_sp.at[didx.at[j0 + 1]],
                                      sem3).wait()
                pltpu.async_copy(g_hbm.at[sidx.at[j0 + 3]], rows1, sem1)

            return carry

        lax.fori_loop(0, PCH // 2, body, 0)
        pltpu.make_async_copy(rows0, acc_sp.at[didx.at[PCH - 2]], sem2).wait()
        pltpu.make_async_copy(rows1, acc_sp.at[didx.at[PCH - 1]], sem3).wait()
        return pcarry

    lax.fori_loop(0, nph, phase, 0)
    plsc.subcore_barrier()

    @pl.when(c == 0)
    def _():
        for i in range(STRIPE // CHUNK):
            r0 = s * STRIPE + i * CHUNK
            pltpu.sync_copy(acc_sp.at[pl.ds(r0, CHUNK)],
                            a0_hbm.at[pl.ds(r0, CHUNK)])

    @pl.when(c == 1)
    def _():
        for i in range(STRIPE // CHUNK):
            r0 = s * STRIPE + i * CHUNK
            pltpu.sync_copy(acc_sp.at[pl.ds(r0, CHUNK)],
                            a1_hbm.at[pl.ds(r0, CHUNK)])


_BLK = 1000


def _tc_linear_body(x_ref, w_ref, b_ref, deg_ref, g_ref):
    h = jnp.dot(x_ref[...], w_ref[...], preferred_element_type=jnp.float32)
    h = h + b_ref[...]
    g_ref[...] = h * lax.rsqrt(deg_ref[...])


def _tc_finish_body(a0_ref, a1_ref, g_ref, deg_ref, out_ref):
    tot = a0_ref[...] + a1_ref[...] + g_ref[...]
    out_ref[...] = jnp.maximum(lax.rsqrt(deg_ref[...]) * tot, 0.0)


_tc_linear = pl.pallas_call(
    _tc_linear_body,
    grid=(N // _BLK,),
    in_specs=[
        pl.BlockSpec((_BLK, D), lambda i: (i, 0)),
        pl.BlockSpec((D, D), lambda i: (0, 0)),
        pl.BlockSpec((1, D), lambda i: (0, 0)),
        pl.BlockSpec((_BLK, 1), lambda i: (i, 0)),
    ],
    out_specs=pl.BlockSpec((_BLK, D), lambda i: (i, 0)),
    out_shape=jax.ShapeDtypeStruct((N, D), jnp.float32),
)

_tc_finish = pl.pallas_call(
    _tc_finish_body,
    grid=(N // _BLK,),
    in_specs=[
        pl.BlockSpec((_BLK, D), lambda i: (i, 0)),
        pl.BlockSpec((_BLK, D), lambda i: (i, 0)),
        pl.BlockSpec((_BLK, D), lambda i: (i, 0)),
        pl.BlockSpec((_BLK, 1), lambda i: (i, 0)),
    ],
    out_specs=pl.BlockSpec((_BLK, D), lambda i: (i, 0)),
    out_shape=jax.ShapeDtypeStruct((N, D), jnp.float32),
)


def kernel(x, edge_index, W, b):
    src = edge_index[0]
    dst = edge_index[1]
    pad = E2 - src.shape[0]
    src_p = jnp.concatenate(
        [src, jnp.zeros((pad,), jnp.int32)]).reshape(E2 // CHUNK, CHUNK)
    dst_p = jnp.concatenate(
        [dst, jnp.full((pad,), PAD_BIN, jnp.int32)]).reshape(E2 // CHUNK, CHUNK)

    d0, d1 = _sc_deg(dst_p)
    deg2 = (d0 + d1)[:N, None] + 1.0

    g = _tc_linear(x, W, b.reshape(1, D), deg2)

    z128 = jnp.zeros((CHUNK, D), jnp.float32)
    a0, a1 = _sc_edge(src_p, dst_p, g, z128)

    return _tc_finish(a0, a1, g, deg2)
